# Initial kernel scaffold; baseline (speedup 1.0000x reference)
import jax
import jax.numpy as jnp
from jax.experimental import pallas as pl


def kernel(x, edge_index, edge_attr, u, batch, W1, b1, W2, b2):
    raise NotImplementedError("write your pallas kernel here")


# speedup vs baseline: 18.7430x; 18.7430x over previous
import jax
import jax.numpy as jnp
from jax.experimental import pallas as pl
from jax.experimental.pallas import tpu as pltpu

N, E, B, H = 10000, 320000, 256, 128
BK = 2560
NB = E // BK


def _body(x_ref, row_ref, e_ref, u_ref, batch_ref, w1_ref, b1_ref, w2_ref,
          b2_ref, out_ref, starts_scr, hist_scr, acc_scr, cnt_scr):
    i = pl.program_id(0)

    @pl.when(i == 0)
    def _init():
        b_iota = jax.lax.broadcasted_iota(jnp.int32, (B, N), 0)
        hist_col = jnp.sum(jnp.equal(batch_ref[...], b_iota).astype(jnp.float32),
                           axis=1, keepdims=True)
        tri = (jax.lax.broadcasted_iota(jnp.int32, (B, B), 0)
               > jax.lax.broadcasted_iota(jnp.int32, (B, B), 1)).astype(jnp.float32)
        starts_col = jnp.dot(tri, hist_col, preferred_element_type=jnp.float32)
        starts_scr[...] = jnp.broadcast_to(starts_col.astype(jnp.int32), (B, H))
        hist_scr[...] = jnp.broadcast_to(hist_col.astype(jnp.int32), (B, H))
        acc_scr[...] = jnp.zeros((B, H), jnp.float32)
        cnt_scr[...] = jnp.zeros((B, H), jnp.float32)

    starts_col = starts_scr[:, 0:1]
    ends_col = starts_col + hist_scr[:, 0:1]

    row2 = row_ref[...].reshape(1, BK)
    mask = (row2 >= starts_col) & (row2 < ends_col)
    onehot = mask.astype(jnp.bfloat16)
    eblk = e_ref[...].astype(jnp.bfloat16)
    acc_scr[...] += jnp.dot(onehot, eblk, preferred_element_type=jnp.float32)
    cnt_col = jnp.sum(mask.astype(jnp.float32), axis=1, keepdims=True)
    cnt_scr[...] += jnp.broadcast_to(cnt_col, (B, H))

    @pl.when(i == NB - 1)
    def _finish():
        s_col = starts_scr[:, 0:1]
        h_col = hist_scr[:, 0:1]
        n_iota = jax.lax.broadcasted_iota(jnp.int32, (B, N), 1)
        maskx = ((n_iota >= s_col) & (n_iota < s_col + h_col)).astype(jnp.float32)
        sum_x = jnp.dot(maskx, x_ref[...], preferred_element_type=jnp.float32)
        x_mean = sum_x / jnp.maximum(h_col.astype(jnp.float32), 1.0)
        e_mean = acc_scr[...] / jnp.maximum(cnt_scr[...], 1.0)
        cat = jnp.concatenate([u_ref[...], x_mean, e_mean], axis=1)
        dn = (((1,), (1,)), ((), ()))
        h1 = jax.lax.dot_general(cat, w1_ref[...], dn,
                                 preferred_element_type=jnp.float32) + b1_ref[...]
        h1 = jnp.maximum(h1, 0.0)
        out_ref[...] = jax.lax.dot_general(h1, w2_ref[...], dn,
                                           preferred_element_type=jnp.float32) + b2_ref[...]


def kernel(x, edge_index, edge_attr, u, batch, W1, b1, W2, b2):
    row3 = edge_index[0].reshape(NB, 1, BK)
    batch2 = batch.reshape(1, N)
    b1r = b1.reshape(1, H)
    b2r = b2.reshape(1, H)
    grid = (NB,)
    return pl.pallas_call(
        _body,
        grid=grid,
        in_specs=[
            pl.BlockSpec((N, H), lambda i: (0, 0)),
            pl.BlockSpec((1, 1, BK), lambda i: (i, 0, 0)),
            pl.BlockSpec((BK, H), lambda i: (i, 0)),
            pl.BlockSpec((B, H), lambda i: (0, 0)),
            pl.BlockSpec((1, N), lambda i: (0, 0)),
            pl.BlockSpec((H, 3 * H), lambda i: (0, 0)),
            pl.BlockSpec((1, H), lambda i: (0, 0)),
            pl.BlockSpec((H, H), lambda i: (0, 0)),
            pl.BlockSpec((1, H), lambda i: (0, 0)),
        ],
        out_specs=pl.BlockSpec((B, H), lambda i: (0, 0)),
        out_shape=jax.ShapeDtypeStruct((B, H), jnp.float32),
        scratch_shapes=[
            pltpu.VMEM((B, H), jnp.int32),
            pltpu.VMEM((B, H), jnp.int32),
            pltpu.VMEM((B, H), jnp.float32),
            pltpu.VMEM((B, H), jnp.float32),
        ],
        compiler_params=pltpu.CompilerParams(
            dimension_semantics=("arbitrary",),
        ),
    )(x, row3, edge_attr, u, batch2, W1, b1r, W2, b2r)
